# baseline (speedup 1.0000x reference)
import functools

import jax
import jax.numpy as jnp
from jax import lax
from jax.experimental import pallas as pl
from jax.experimental.pallas import tpu as pltpu
from jax.experimental.pallas import tpu_sc as plsc

N_NODES = 10000
N_EDGES = 320000
D = 128

NC = 2
NS = 16
NW = NC * NS
EPW = N_EDGES // NW
C = 40
NCHUNK = EPW // C
NPAIR = NCHUNK // 2
N_PAD = 10240
RPS = N_PAD // NS


def _sc_body(node_hbm, edge_hbm, src_hbm, dst_hbm, out_hbm,
             src_all, dst_all, rows0, rows1, erows0, erows1, acc,
             isem, gsem0, gsem1, esem0, esem1, ssem0, ssem1):
    cid = lax.axis_index("c")
    sid = lax.axis_index("s")
    wid = sid * NC + cid

    rows = (rows0, rows1)
    erows = (erows0, erows1)
    gsem = (gsem0, gsem1)
    esem = (esem0, esem1)
    ssem = (ssem0, ssem1)

    pltpu.async_copy(src_hbm.at[wid], src_all, isem)
    pltpu.async_copy(dst_hbm.at[wid], dst_all, isem)

    z = jnp.zeros((16,), jnp.float32)

    def zero_row(r, _):
        for j in range(D // 16):
            rows0[r, pl.ds(j * 16, 16)] = z
        return 0

    lax.fori_loop(0, C, zero_row, 0)
    for k in range(RPS // C):
        pltpu.sync_copy(rows0, acc.at[pl.ds(sid * RPS + k * C, C), :])
    plsc.subcore_barrier()

    pltpu.make_async_copy(src_hbm.at[wid], src_all, isem).wait()
    pltpu.make_async_copy(dst_hbm.at[wid], dst_all, isem).wait()

    def issue_fetch(t, b):
        base = wid * EPW + t * C
        pltpu.async_copy(node_hbm.at[src_all.at[pl.ds(t * C, C)]],
                         rows[b], gsem[b])
        pltpu.async_copy(edge_hbm.at[pl.ds(base, C), :], erows[b], esem[b])

    def wait_fetch(b):
        pltpu.make_async_copy(node_hbm.at[pl.ds(0, C), :], rows[b],
                              gsem[b]).wait()
        pltpu.make_async_copy(edge_hbm.at[pl.ds(0, C), :], erows[b],
                              esem[b]).wait()

    def issue_scatter(t, b):
        def add_row(r, _):
            for j in range(D // 16):
                s = pl.ds(j * 16, 16)
                rows[b][r, s] = rows[b][r, s] + erows[b][r, s]
            return 0

        lax.fori_loop(0, C, add_row, 0)
        idx = dst_all.at[pl.ds(t * C, C)]
        pltpu.async_copy(rows[b], acc.at[idx], ssem[b], add=True)

    def wait_scatter(b):
        pltpu.make_async_copy(rows[b], acc.at[pl.ds(0, C), :], ssem[b]).wait()

    issue_fetch(0, 0)

    def pair(k, _):
        t0 = 2 * k
        wait_fetch(0)
        issue_scatter(t0, 0)

        @pl.when(k > 0)
        def _():
            wait_scatter(1)

        issue_fetch(t0 + 1, 1)
        wait_fetch(1)
        issue_scatter(t0 + 1, 1)
        wait_scatter(0)

        @pl.when(k < NPAIR - 1)
        def _():
            issue_fetch(t0 + 2, 0)

        return 0

    lax.fori_loop(0, NPAIR, pair, 0)
    wait_scatter(1)
    plsc.subcore_barrier()

    pltpu.sync_copy(acc.at[pl.ds(sid * RPS, RPS), :],
                    out_hbm.at[cid, pl.ds(sid * RPS, RPS), :])


@jax.jit
def _sc_aggregate(node_feats, edge_feats, src_r, dst_r):
    mesh = plsc.VectorSubcoreMesh(core_axis_name="c", subcore_axis_name="s")
    f = pl.kernel(
        _sc_body,
        out_type=jax.ShapeDtypeStruct((NC, N_PAD, D), jnp.float32),
        mesh=mesh,
        scratch_types=[
            pltpu.VMEM((EPW,), jnp.int32),
            pltpu.VMEM((EPW,), jnp.int32),
            pltpu.VMEM((C, D), jnp.float32),
            pltpu.VMEM((C, D), jnp.float32),
            pltpu.VMEM((C, D), jnp.float32),
            pltpu.VMEM((C, D), jnp.float32),
            pltpu.VMEM_SHARED((N_PAD, D), jnp.float32),
            pltpu.SemaphoreType.DMA,
            pltpu.SemaphoreType.DMA,
            pltpu.SemaphoreType.DMA,
            pltpu.SemaphoreType.DMA,
            pltpu.SemaphoreType.DMA,
            pltpu.SemaphoreType.DMA,
            pltpu.SemaphoreType.DMA,
        ],
    )
    return f(node_feats, edge_feats, src_r, dst_r)


def _mlp_body(p_ref, w1_ref, b1_ref, w2_ref, b2_ref, o_ref):
    h = p_ref[0] + p_ref[1]
    a = jnp.dot(h, w1_ref[...], preferred_element_type=jnp.float32)
    a = jnp.maximum(a + b1_ref[...], 0.0)
    o = jnp.dot(a, w2_ref[...], preferred_element_type=jnp.float32)
    o_ref[...] = o + b2_ref[...]


@jax.jit
def _mlp(partials, W1, b1, W2, b2):
    B = 1000
    grid = (N_NODES // B,)
    return pl.pallas_call(
        _mlp_body,
        grid=grid,
        in_specs=[
            pl.BlockSpec((NC, B, D), lambda i: (0, i, 0)),
            pl.BlockSpec((D, 2 * D), lambda i: (0, 0)),
            pl.BlockSpec((1, 2 * D), lambda i: (0, 0)),
            pl.BlockSpec((2 * D, D), lambda i: (0, 0)),
            pl.BlockSpec((1, D), lambda i: (0, 0)),
        ],
        out_specs=pl.BlockSpec((B, D), lambda i: (i, 0)),
        out_shape=jax.ShapeDtypeStruct((N_NODES, D), jnp.float32),
    )(partials, W1, b1, W2, b2)


def kernel(node_feats, edge_feats, edge_index, W1, b1, W2, b2):
    ei = edge_index.astype(jnp.int32)
    src_r = ei[0].reshape(NW, EPW)
    dst_r = ei[1].reshape(NW, EPW)
    partials = _sc_aggregate(node_feats, edge_feats, src_r, dst_r)
    return _mlp(partials, W1, b1.reshape(1, -1), W2, b2.reshape(1, -1))


# speedup vs baseline: 1.2560x; 1.2560x over previous
import functools

import jax
import jax.numpy as jnp
from jax import lax
from jax.experimental import pallas as pl
from jax.experimental.pallas import tpu as pltpu
from jax.experimental.pallas import tpu_sc as plsc

N_NODES = 10000
N_EDGES = 320000
D = 128

NC = 2
NS = 16
NW = NC * NS
EPW = N_EDGES // NW
C = 40
NCHUNK = EPW // C
NPAIR = NCHUNK // 2
N_PAD = 10240
RPS = N_PAD // NS


def _sc_body(node_hbm, edge_hbm, src_hbm, dst_hbm, out_hbm,
             src_all, dst_all, rows0, rows1, erows0, erows1, acc,
             isem, gsem0, gsem1, esem0, esem1, ssem0, ssem1):
    cid = lax.axis_index("c")
    sid = lax.axis_index("s")
    wid = sid * NC + cid

    rows = (rows0, rows1)
    erows = (erows0, erows1)
    gsem = (gsem0, gsem1)
    esem = (esem0, esem1)
    ssem = (ssem0, ssem1)

    pltpu.async_copy(src_hbm.at[wid], src_all, isem)
    pltpu.async_copy(dst_hbm.at[wid], dst_all, isem)

    z = jnp.zeros((16,), jnp.float32)

    def zero_row(r, _):
        for j in range(D // 16):
            rows0[r, pl.ds(j * 16, 16)] = z
        return 0

    lax.fori_loop(0, C, zero_row, 0)
    for k in range(RPS // C):
        pltpu.sync_copy(rows0, acc.at[pl.ds(sid * RPS + k * C, C), :])
    plsc.subcore_barrier()

    pltpu.make_async_copy(src_hbm.at[wid], src_all, isem).wait()
    pltpu.make_async_copy(dst_hbm.at[wid], dst_all, isem).wait()

    def issue_fetch(t, b):
        base = wid * EPW + t * C
        pltpu.async_copy(node_hbm.at[src_all.at[pl.ds(t * C, C)]],
                         rows[b], gsem[b])
        pltpu.async_copy(edge_hbm.at[pl.ds(base, C), :], erows[b], esem[b])

    def wait_fetch(b):
        pltpu.make_async_copy(node_hbm.at[pl.ds(0, C), :], rows[b],
                              gsem[b]).wait()
        pltpu.make_async_copy(edge_hbm.at[pl.ds(0, C), :], erows[b],
                              esem[b]).wait()

    def issue_scatter(t, b):
        idx = dst_all.at[pl.ds(t * C, C)]
        pltpu.async_copy(rows[b], acc.at[idx], ssem[b], add=True)
        pltpu.async_copy(erows[b], acc.at[idx], ssem[b], add=True)

    def wait_scatter(b):
        pltpu.make_async_copy(rows[b], acc.at[pl.ds(0, C), :], ssem[b]).wait()
        pltpu.make_async_copy(erows[b], acc.at[pl.ds(0, C), :], ssem[b]).wait()

    issue_fetch(0, 0)

    def pair(k, _):
        t0 = 2 * k
        wait_fetch(0)
        issue_scatter(t0, 0)

        @pl.when(k > 0)
        def _():
            wait_scatter(1)

        issue_fetch(t0 + 1, 1)
        wait_fetch(1)
        issue_scatter(t0 + 1, 1)
        wait_scatter(0)

        @pl.when(k < NPAIR - 1)
        def _():
            issue_fetch(t0 + 2, 0)

        return 0

    lax.fori_loop(0, NPAIR, pair, 0)
    wait_scatter(1)
    plsc.subcore_barrier()

    pltpu.sync_copy(acc.at[pl.ds(sid * RPS, RPS), :],
                    out_hbm.at[cid, pl.ds(sid * RPS, RPS), :])


@jax.jit
def _sc_aggregate(node_feats, edge_feats, src_r, dst_r):
    mesh = plsc.VectorSubcoreMesh(core_axis_name="c", subcore_axis_name="s")
    f = pl.kernel(
        _sc_body,
        out_type=jax.ShapeDtypeStruct((NC, N_PAD, D), jnp.float32),
        mesh=mesh,
        scratch_types=[
            pltpu.VMEM((EPW,), jnp.int32),
            pltpu.VMEM((EPW,), jnp.int32),
            pltpu.VMEM((C, D), jnp.float32),
            pltpu.VMEM((C, D), jnp.float32),
            pltpu.VMEM((C, D), jnp.float32),
            pltpu.VMEM((C, D), jnp.float32),
            pltpu.VMEM_SHARED((N_PAD, D), jnp.float32),
            pltpu.SemaphoreType.DMA,
            pltpu.SemaphoreType.DMA,
            pltpu.SemaphoreType.DMA,
            pltpu.SemaphoreType.DMA,
            pltpu.SemaphoreType.DMA,
            pltpu.SemaphoreType.DMA,
            pltpu.SemaphoreType.DMA,
        ],
    )
    return f(node_feats, edge_feats, src_r, dst_r)


def _mlp_body(p_ref, w1_ref, b1_ref, w2_ref, b2_ref, o_ref):
    h = p_ref[0] + p_ref[1]
    a = jnp.dot(h, w1_ref[...], preferred_element_type=jnp.float32)
    a = jnp.maximum(a + b1_ref[...], 0.0)
    o = jnp.dot(a, w2_ref[...], preferred_element_type=jnp.float32)
    o_ref[...] = o + b2_ref[...]


@jax.jit
def _mlp(partials, W1, b1, W2, b2):
    B = 1000
    grid = (N_NODES // B,)
    return pl.pallas_call(
        _mlp_body,
        grid=grid,
        in_specs=[
            pl.BlockSpec((NC, B, D), lambda i: (0, i, 0)),
            pl.BlockSpec((D, 2 * D), lambda i: (0, 0)),
            pl.BlockSpec((1, 2 * D), lambda i: (0, 0)),
            pl.BlockSpec((2 * D, D), lambda i: (0, 0)),
            pl.BlockSpec((1, D), lambda i: (0, 0)),
        ],
        out_specs=pl.BlockSpec((B, D), lambda i: (i, 0)),
        out_shape=jax.ShapeDtypeStruct((N_NODES, D), jnp.float32),
    )(partials, W1, b1, W2, b2)


def kernel(node_feats, edge_feats, edge_index, W1, b1, W2, b2):
    ei = edge_index.astype(jnp.int32)
    src_r = ei[0].reshape(NW, EPW)
    dst_r = ei[1].reshape(NW, EPW)
    partials = _sc_aggregate(node_feats, edge_feats, src_r, dst_r)
    return _mlp(partials, W1, b1.reshape(1, -1), W2, b2.reshape(1, -1))
